# Initial kernel scaffold; baseline (speedup 1.0000x reference)
import jax
import jax.numpy as jnp
from jax.experimental import pallas as pl


def kernel(input, target):
    raise NotImplementedError("write your pallas kernel here")


# speedup vs baseline: 35.7806x; 35.7806x over previous
import functools

import jax
import jax.numpy as jnp
from jax import lax
from jax.experimental import pallas as pl
from jax.experimental.pallas import tpu as pltpu

N = 8192
D = 64
K = 64
BLOCK_R = 512
T_BISECT = 24
GRID = N // BLOCK_R


def _loss_body(pred_ref, target_ref, out_ref, tn_ref, s_ref):
    b = pl.program_id(0)

    @pl.when(b == 0)
    def _():
        t = target_ref[...]
        nrm = jnp.sqrt(jnp.sum(t * t, axis=1, keepdims=True))
        tn_ref[...] = t / jnp.maximum(nrm, 1e-12)

    p = pred_ref[...]
    nrm = jnp.sqrt(jnp.sum(p * p, axis=1, keepdims=True))
    pn = p / jnp.maximum(nrm, 1e-12)
    tn = tn_ref[...]

    s = lax.dot_general(pn, tn, (((1,), (1,)), ((), ())),
                        preferred_element_type=jnp.float32)

    tnb = tn_ref[pl.ds(b * BLOCK_R, BLOCK_R), :]
    s_ii = jnp.sum(pn * tnb, axis=1, keepdims=True)

    row = lax.broadcasted_iota(jnp.int32, (BLOCK_R, N), 0)
    col = lax.broadcasted_iota(jnp.int32, (BLOCK_R, N), 1)
    diag = col == row + b * BLOCK_R
    s_ref[...] = jnp.where(diag, jnp.nan, s)

    kf = jnp.float32(K)

    def bisect(_, carry):
        lo, hi, lo2, hi2 = carry
        sv = s_ref[...]
        mid = 0.5 * (lo + hi)
        cnt = jnp.sum(jnp.where(sv > mid, 1.0, 0.0), axis=1, keepdims=True)
        ge = cnt >= kf
        lo = jnp.where(ge, mid, lo)
        hi = jnp.where(ge, hi, mid)
        mid2 = 0.5 * (lo2 + hi2)
        cnt2 = jnp.sum(jnp.where(sv < mid2, 1.0, 0.0), axis=1, keepdims=True)
        ge2 = cnt2 >= kf
        hi2 = jnp.where(ge2, mid2, hi2)
        lo2 = jnp.where(ge2, lo2, mid2)
        return lo, hi, lo2, hi2

    ones = jnp.ones((BLOCK_R, 1), jnp.float32)
    lo, hi, lo2, hi2 = lax.fori_loop(
        0, T_BISECT, bisect,
        (-1.01 * ones, 1.01 * ones, -1.01 * ones, 1.01 * ones))

    sv = s_ref[...]
    t1 = lo
    m1 = sv > t1
    cnt1 = jnp.sum(jnp.where(m1, 1.0, 0.0), axis=1, keepdims=True)
    sum1 = jnp.sum(jnp.where(m1, sv, 0.0), axis=1, keepdims=True)
    s_top = sum1 + t1 * (kf - cnt1)

    t2 = hi2
    m2 = sv < t2
    cnt2 = jnp.sum(jnp.where(m2, 1.0, 0.0), axis=1, keepdims=True)
    sum2 = jnp.sum(jnp.where(m2, sv, 0.0), axis=1, keepdims=True)
    s_bot = sum2 + t2 * (kf - cnt2)

    sum_dist = (2.0 * kf - s_top - s_bot) * 0.5
    dist_ap = (1.0 - s_ii) * 0.5
    positive_risk = 0.5 * dist_ap
    negative_risk = -(0.5 / (2.0 * kf)) * sum_dist
    loss_n = jnp.where(negative_risk < 0.0, -negative_risk,
                       positive_risk + negative_risk)
    blk = jnp.sum(loss_n, axis=0, keepdims=True) * (1.0 / N)

    @pl.when(b == 0)
    def _():
        out_ref[...] = jnp.zeros_like(out_ref)

    out_ref[...] += blk


@jax.jit
def kernel(input, target):
    out = pl.pallas_call(
        _loss_body,
        grid=(GRID,),
        in_specs=[
            pl.BlockSpec((BLOCK_R, D), lambda b: (b, 0)),
            pl.BlockSpec((N, D), lambda b: (0, 0)),
        ],
        out_specs=pl.BlockSpec((1, 1), lambda b: (0, 0)),
        out_shape=jax.ShapeDtypeStruct((1, 1), jnp.float32),
        scratch_shapes=[
            pltpu.VMEM((N, D), jnp.float32),
            pltpu.VMEM((BLOCK_R, N), jnp.float32),
        ],
        compiler_params=pltpu.CompilerParams(
            dimension_semantics=("arbitrary",),
        ),
    )(input, target)
    return out[0, 0]


# speedup vs baseline: 51.0104x; 1.4256x over previous
import functools

import jax
import jax.numpy as jnp
from jax import lax
from jax.experimental import pallas as pl
from jax.experimental.pallas import tpu as pltpu

N = 8192
D = 64
K = 64
BLOCK_R = 512
T_BISECT = 16
GRID = N // BLOCK_R


def _loss_body(pred_ref, target_ref, out_ref, tn_ref, s_ref):
    b = pl.program_id(0)

    @pl.when(b == 0)
    def _():
        t = target_ref[...]
        nrm = jnp.sqrt(jnp.sum(t * t, axis=1, keepdims=True))
        tn_ref[...] = t / jnp.maximum(nrm, 1e-12)

    p = pred_ref[...]
    nrm = jnp.sqrt(jnp.sum(p * p, axis=1, keepdims=True))
    pn = p / jnp.maximum(nrm, 1e-12)
    tn = tn_ref[...]

    s = lax.dot_general(pn, tn, (((1,), (1,)), ((), ())),
                        preferred_element_type=jnp.float32)

    tnb = tn_ref[pl.ds(b * BLOCK_R, BLOCK_R), :]
    s_ii = jnp.sum(pn * tnb, axis=1, keepdims=True)

    row = lax.broadcasted_iota(jnp.int32, (BLOCK_R, N), 0)
    col = lax.broadcasted_iota(jnp.int32, (BLOCK_R, N), 1)
    diag = col == row + b * BLOCK_R
    s_ref[...] = jnp.where(diag, jnp.nan, s)

    kf = jnp.float32(K)

    def bisect(_, carry):
        lo, hi, lo2, hi2 = carry
        sv = s_ref[...]
        mid = 0.5 * (lo + hi)
        cnt = jnp.sum(jnp.where(sv > mid, 1.0, 0.0), axis=1, keepdims=True)
        ge = cnt >= kf
        lo = jnp.where(ge, mid, lo)
        hi = jnp.where(ge, hi, mid)
        mid2 = 0.5 * (lo2 + hi2)
        cnt2 = jnp.sum(jnp.where(sv < mid2, 1.0, 0.0), axis=1, keepdims=True)
        ge2 = cnt2 >= kf
        hi2 = jnp.where(ge2, mid2, hi2)
        lo2 = jnp.where(ge2, lo2, mid2)
        return lo, hi, lo2, hi2

    ones = jnp.ones((BLOCK_R, 1), jnp.float32)
    lo, hi, lo2, hi2 = lax.fori_loop(
        0, T_BISECT, bisect,
        (-1.01 * ones, 1.01 * ones, -1.01 * ones, 1.01 * ones))

    sv = s_ref[...]
    t1 = lo
    m1 = sv > t1
    cnt1 = jnp.sum(jnp.where(m1, 1.0, 0.0), axis=1, keepdims=True)
    sum1 = jnp.sum(jnp.where(m1, sv, 0.0), axis=1, keepdims=True)
    s_top = sum1 + t1 * (kf - cnt1)

    t2 = hi2
    m2 = sv < t2
    cnt2 = jnp.sum(jnp.where(m2, 1.0, 0.0), axis=1, keepdims=True)
    sum2 = jnp.sum(jnp.where(m2, sv, 0.0), axis=1, keepdims=True)
    s_bot = sum2 + t2 * (kf - cnt2)

    sum_dist = (2.0 * kf - s_top - s_bot) * 0.5
    dist_ap = (1.0 - s_ii) * 0.5
    positive_risk = 0.5 * dist_ap
    negative_risk = -(0.5 / (2.0 * kf)) * sum_dist
    loss_n = jnp.where(negative_risk < 0.0, -negative_risk,
                       positive_risk + negative_risk)
    blk = jnp.sum(loss_n, axis=0, keepdims=True) * (1.0 / N)

    @pl.when(b == 0)
    def _():
        out_ref[...] = jnp.zeros_like(out_ref)

    out_ref[...] += blk


@jax.jit
def kernel(input, target):
    out = pl.pallas_call(
        _loss_body,
        grid=(GRID,),
        in_specs=[
            pl.BlockSpec((BLOCK_R, D), lambda b: (b, 0)),
            pl.BlockSpec((N, D), lambda b: (0, 0)),
        ],
        out_specs=pl.BlockSpec((1, 1), lambda b: (0, 0)),
        out_shape=jax.ShapeDtypeStruct((1, 1), jnp.float32),
        scratch_shapes=[
            pltpu.VMEM((N, D), jnp.float32),
            pltpu.VMEM((BLOCK_R, N), jnp.float32),
        ],
        compiler_params=pltpu.CompilerParams(
            dimension_semantics=("arbitrary",),
        ),
    )(input, target)
    return out[0, 0]
